# baseline (speedup 1.0000x reference)
import jax
import jax.numpy as jnp
from jax import lax
from jax.experimental import pallas as pl
from jax.experimental.pallas import tpu as pltpu
from jax.experimental.pallas import tpu_sc as plsc

NCORES = 2
NSUB = 16
NTILES = NCORES * NSUB
CHUNK = 80


def _sc_aggregate(x, src, dst, n_pad, n_chunks):
    D = x.shape[1]
    rows_per_tile = n_pad // NSUB
    n_zero_copies = rows_per_tile // CHUNK
    zero_rem = rows_per_tile % CHUNK

    def body(x_hbm, src_hbm, dst_hbm, agg_out, cnt_out,
             acc, cnta, idx_c, rows, ones_v, zc, isem, gsem, csem, ssem):
        c = lax.axis_index("c")
        s = lax.axis_index("s")
        w = c * NSUB + s
        ept = n_chunks * CHUNK
        n_edges = NTILES * ept
        ebase = w * ept

        def fetch_idx(j, slot):
            off = ebase + j * CHUNK
            pltpu.async_copy(src_hbm.at[pl.ds(off, CHUNK)],
                             idx_c.at[slot, 0], isem)
            pltpu.async_copy(dst_hbm.at[pl.ds(off, CHUNK)],
                             idx_c.at[slot, 1], isem)

        def wait_idx():
            pltpu.make_async_copy(src_hbm.at[pl.ds(0, CHUNK)],
                                  idx_c.at[0, 0], isem).wait()
            pltpu.make_async_copy(src_hbm.at[pl.ds(0, CHUNK)],
                                  idx_c.at[0, 1], isem).wait()

        def fill_rows(i, carry):
            r = i // (D // 16)
            k = i % (D // 16)
            rows[0, r, pl.ds(k * 16, 16)] = jnp.zeros((16,), jnp.float32)
            return carry

        lax.fori_loop(0, CHUNK * (D // 16), fill_rows, 0)

        def fill_ones(i, carry):
            ones_v[pl.ds(i * 16, 16)] = jnp.ones((16,), jnp.float32)
            return carry

        lax.fori_loop(0, CHUNK // 16, fill_ones, 0)

        def fill_zc(i, carry):
            zc[pl.ds(i * 16, 16)] = jnp.zeros((16,), jnp.float32)
            return carry

        lax.fori_loop(0, rows_per_tile // 16, fill_zc, 0)

        base = s * rows_per_tile
        for k in range(n_zero_copies):
            pltpu.sync_copy(rows.at[0], acc.at[pl.ds(base + k * CHUNK, CHUNK)])
        if zero_rem:
            pltpu.sync_copy(rows.at[0, pl.ds(0, zero_rem)],
                            acc.at[pl.ds(base + n_zero_copies * CHUNK, zero_rem)])
        pltpu.sync_copy(zc, cnta.at[pl.ds(base, rows_per_tile)])
        plsc.subcore_barrier()

        fetch_idx(0, 0)
        fetch_idx(1, 1)
        fetch_idx(2, 2)
        wait_idx()
        pltpu.async_copy(x_hbm.at[idx_c.at[0, 0]], rows.at[0], gsem)
        wait_idx()
        pltpu.async_copy(x_hbm.at[idx_c.at[1, 0]], rows.at[1], gsem)

        def wait_scatter():
            pltpu.make_async_copy(rows.at[0], acc.at[idx_c.at[0, 1]], ssem).wait()

        def wait_count():
            pltpu.make_async_copy(ones_v, cnta.at[idx_c.at[0, 1]], csem).wait()

        def chunk_body(j, carry):
            buf = lax.rem(j, 3)
            sl = lax.rem(j, 4)
            gbuf = lax.rem(j + 2, 3)
            gsl = lax.rem(j + 2, 4)
            psl = lax.rem(j + 3, 4)

            @pl.when(j >= 1)
            def _():
                wait_scatter()
                wait_count()

            @pl.when(j + 3 < n_chunks)
            def _():
                fetch_idx(j + 3, psl)

            @pl.when(j + 2 < n_chunks)
            def _():
                wait_idx()
                pltpu.async_copy(x_hbm.at[idx_c.at[gsl, 0]], rows.at[gbuf], gsem)

            pltpu.make_async_copy(x_hbm.at[idx_c.at[sl, 0]], rows.at[buf], gsem).wait()
            pltpu.async_copy(ones_v, cnta.at[idx_c.at[sl, 1]], csem, add=True)
            pltpu.async_copy(rows.at[buf], acc.at[idx_c.at[sl, 1]], ssem, add=True)
            return carry

        lax.fori_loop(0, n_chunks - 1, chunk_body, 0)

        jl = n_chunks - 1
        buf = (jl) % 3
        sl = (jl) % 4
        wait_scatter()
        wait_count()
        pltpu.make_async_copy(x_hbm.at[idx_c.at[sl, 0]], rows.at[buf], gsem).wait()
        pltpu.async_copy(ones_v, cnta.at[idx_c.at[sl, 1]], csem, add=True)
        pltpu.async_copy(rows.at[buf], acc.at[idx_c.at[sl, 1]], ssem, add=True)
        wait_scatter()
        wait_count()
        plsc.subcore_barrier()

        pltpu.sync_copy(acc.at[pl.ds(base, rows_per_tile)],
                        agg_out.at[c, pl.ds(base, rows_per_tile)])
        pltpu.sync_copy(cnta.at[pl.ds(base, rows_per_tile)],
                        cnt_out.at[c, pl.ds(base, rows_per_tile)])

    run = pl.kernel(
        body,
        out_type=(jax.ShapeDtypeStruct((NCORES, n_pad, D), jnp.float32),
                  jax.ShapeDtypeStruct((NCORES, n_pad), jnp.float32)),
        mesh=plsc.VectorSubcoreMesh(core_axis_name="c", subcore_axis_name="s"),
        scratch_types=(
            pltpu.VMEM_SHARED((n_pad, D), jnp.float32),
            pltpu.VMEM_SHARED((n_pad,), jnp.float32),
            pltpu.VMEM((4, 2, CHUNK), jnp.int32),
            pltpu.VMEM((3, CHUNK, D), jnp.float32),
            pltpu.VMEM((CHUNK,), jnp.float32),
            pltpu.VMEM((n_pad // NSUB,), jnp.float32),
            pltpu.SemaphoreType.DMA,
            pltpu.SemaphoreType.DMA,
            pltpu.SemaphoreType.DMA,
            pltpu.SemaphoreType.DMA,
        ),
    )
    return run(x, src, dst)


def _tc_combine(agg, cnt, x, wlT, wrT, b):
    N, D = x.shape
    BT = 1000

    def body(agg_ref, cnt_ref, x_ref, wl_ref, wr_ref, b_ref, out_ref):
        p = agg_ref[0] + agg_ref[1]
        cnt_col = cnt_ref[...]
        mean = p / jnp.maximum(cnt_col, 1.0)
        acc = jnp.dot(mean, wl_ref[...], preferred_element_type=jnp.float32)
        acc = acc + jnp.dot(x_ref[...], wr_ref[...], preferred_element_type=jnp.float32)
        out_ref[...] = jnp.maximum(acc + b_ref[...], 0.0)

    return pl.pallas_call(
        body,
        out_shape=jax.ShapeDtypeStruct((N, D), jnp.float32),
        grid=(N // BT,),
        in_specs=[
            pl.BlockSpec((NCORES, BT, D), lambda i: (0, i, 0)),
            pl.BlockSpec((BT, 1), lambda i: (i, 0)),
            pl.BlockSpec((BT, D), lambda i: (i, 0)),
            pl.BlockSpec((D, D), lambda i: (0, 0)),
            pl.BlockSpec((D, D), lambda i: (0, 0)),
            pl.BlockSpec((1, D), lambda i: (0, 0)),
        ],
        out_specs=pl.BlockSpec((BT, D), lambda i: (i, 0)),
    )(agg, cnt, x, wlT, wrT, b)


def kernel(x, edge_index, W_l, b_l, W_r):
    N, D = x.shape
    E = edge_index.shape[1]

    ept = E // NTILES
    n_chunks = ept // CHUNK
    n_pad = -(-(N + 1) // (NSUB * 16)) * (NSUB * 16)

    agg, cnt = _sc_aggregate(x, edge_index[0], edge_index[1], n_pad, n_chunks)
    return _tc_combine(agg, (cnt[0] + cnt[1]).reshape(n_pad, 1), x,
                       W_l.T, W_r.T, b_l.reshape(1, D))


# speedup vs baseline: 1.0718x; 1.0718x over previous
import jax
import jax.numpy as jnp
from jax import lax
from jax.experimental import pallas as pl
from jax.experimental.pallas import tpu as pltpu
from jax.experimental.pallas import tpu_sc as plsc

NCORES = 2
NSUB = 16
NTILES = NCORES * NSUB
CHUNK = 80


def _sc_aggregate(x, ei, n_pad, n_chunks):
    D = x.shape[1]
    rows_per_tile = n_pad // NSUB
    n_zero_copies = rows_per_tile // CHUNK
    zero_rem = rows_per_tile % CHUNK

    def body(x_hbm, ei_hbm, agg_out, cnt_out,
             acc, cnta, idx_c, rows, ones_v, zc, isem, gsem, csem, ssem):
        c = lax.axis_index("c")
        s = lax.axis_index("s")
        w = c * NSUB + s
        ept = n_chunks * CHUNK
        n_edges = NTILES * ept
        ebase = w * ept

        def fetch_idx(j, slot):
            off = ebase + j * CHUNK
            pltpu.async_copy(ei_hbm.at[pl.ds(off, CHUNK)],
                             idx_c.at[slot, 0], isem)
            pltpu.async_copy(ei_hbm.at[pl.ds(n_edges + off, CHUNK)],
                             idx_c.at[slot, 1], isem)

        def wait_idx():
            pltpu.make_async_copy(ei_hbm.at[pl.ds(0, CHUNK)],
                                  idx_c.at[0, 0], isem).wait()
            pltpu.make_async_copy(ei_hbm.at[pl.ds(0, CHUNK)],
                                  idx_c.at[0, 1], isem).wait()

        def fill_rows(i, carry):
            r = i // (D // 16)
            k = i % (D // 16)
            rows[0, r, pl.ds(k * 16, 16)] = jnp.zeros((16,), jnp.float32)
            return carry

        lax.fori_loop(0, CHUNK * (D // 16), fill_rows, 0)

        def fill_ones(i, carry):
            ones_v[pl.ds(i * 16, 16)] = jnp.ones((16,), jnp.float32)
            return carry

        lax.fori_loop(0, CHUNK // 16, fill_ones, 0)

        def fill_zc(i, carry):
            zc[pl.ds(i * 16, 16)] = jnp.zeros((16,), jnp.float32)
            return carry

        lax.fori_loop(0, rows_per_tile // 16, fill_zc, 0)

        base = s * rows_per_tile
        for k in range(n_zero_copies):
            pltpu.sync_copy(rows.at[0], acc.at[pl.ds(base + k * CHUNK, CHUNK)])
        if zero_rem:
            pltpu.sync_copy(rows.at[0, pl.ds(0, zero_rem)],
                            acc.at[pl.ds(base + n_zero_copies * CHUNK, zero_rem)])
        pltpu.sync_copy(zc, cnta.at[pl.ds(base, rows_per_tile)])
        plsc.subcore_barrier()

        fetch_idx(0, 0)
        fetch_idx(1, 1)
        fetch_idx(2, 2)
        wait_idx()
        pltpu.async_copy(x_hbm.at[idx_c.at[0, 0]], rows.at[0], gsem)
        wait_idx()
        pltpu.async_copy(x_hbm.at[idx_c.at[1, 0]], rows.at[1], gsem)

        def wait_scatter():
            pltpu.make_async_copy(rows.at[0], acc.at[idx_c.at[0, 1]], ssem).wait()

        def wait_count():
            pltpu.make_async_copy(ones_v, cnta.at[idx_c.at[0, 1]], csem).wait()

        def chunk_body(j, carry):
            buf = lax.rem(j, 3)
            sl = lax.rem(j, 4)
            gbuf = lax.rem(j + 2, 3)
            gsl = lax.rem(j + 2, 4)
            psl = lax.rem(j + 3, 4)

            @pl.when(j >= 1)
            def _():
                wait_scatter()
                wait_count()

            @pl.when(j + 3 < n_chunks)
            def _():
                fetch_idx(j + 3, psl)

            @pl.when(j + 2 < n_chunks)
            def _():
                wait_idx()
                pltpu.async_copy(x_hbm.at[idx_c.at[gsl, 0]], rows.at[gbuf], gsem)

            pltpu.make_async_copy(x_hbm.at[idx_c.at[sl, 0]], rows.at[buf], gsem).wait()
            pltpu.async_copy(ones_v, cnta.at[idx_c.at[sl, 1]], csem, add=True)
            pltpu.async_copy(rows.at[buf], acc.at[idx_c.at[sl, 1]], ssem, add=True)
            return carry

        lax.fori_loop(0, n_chunks - 1, chunk_body, 0)

        jl = n_chunks - 1
        buf = (jl) % 3
        sl = (jl) % 4
        wait_scatter()
        wait_count()
        pltpu.make_async_copy(x_hbm.at[idx_c.at[sl, 0]], rows.at[buf], gsem).wait()
        pltpu.async_copy(ones_v, cnta.at[idx_c.at[sl, 1]], csem, add=True)
        pltpu.async_copy(rows.at[buf], acc.at[idx_c.at[sl, 1]], ssem, add=True)
        wait_scatter()
        wait_count()
        plsc.subcore_barrier()

        pltpu.sync_copy(acc.at[pl.ds(base, rows_per_tile)],
                        agg_out.at[c, pl.ds(base, rows_per_tile)])
        pltpu.sync_copy(cnta.at[pl.ds(base, rows_per_tile)],
                        cnt_out.at[c, pl.ds(base, rows_per_tile)])

    run = pl.kernel(
        body,
        out_type=(jax.ShapeDtypeStruct((NCORES, n_pad, D), jnp.float32),
                  jax.ShapeDtypeStruct((NCORES, n_pad), jnp.float32)),
        mesh=plsc.VectorSubcoreMesh(core_axis_name="c", subcore_axis_name="s"),
        scratch_types=(
            pltpu.VMEM_SHARED((n_pad, D), jnp.float32),
            pltpu.VMEM_SHARED((n_pad,), jnp.float32),
            pltpu.VMEM((4, 2, CHUNK), jnp.int32),
            pltpu.VMEM((3, CHUNK, D), jnp.float32),
            pltpu.VMEM((CHUNK,), jnp.float32),
            pltpu.VMEM((n_pad // NSUB,), jnp.float32),
            pltpu.SemaphoreType.DMA,
            pltpu.SemaphoreType.DMA,
            pltpu.SemaphoreType.DMA,
            pltpu.SemaphoreType.DMA,
        ),
    )
    return run(x, ei)


def _tc_combine(agg, cnt, x, wlT, wrT, b):
    N, D = x.shape
    BT = 1000

    def body(agg_ref, cnt_ref, x_ref, wl_ref, wr_ref, b_ref, out_ref):
        p = agg_ref[0] + agg_ref[1]
        cnt_col = cnt_ref[...]
        mean = p / jnp.maximum(cnt_col, 1.0)
        acc = jnp.dot(mean, wl_ref[...], preferred_element_type=jnp.float32)
        acc = acc + jnp.dot(x_ref[...], wr_ref[...], preferred_element_type=jnp.float32)
        out_ref[...] = jnp.maximum(acc + b_ref[...], 0.0)

    return pl.pallas_call(
        body,
        out_shape=jax.ShapeDtypeStruct((N, D), jnp.float32),
        grid=(N // BT,),
        in_specs=[
            pl.BlockSpec((NCORES, BT, D), lambda i: (0, i, 0)),
            pl.BlockSpec((BT, 1), lambda i: (i, 0)),
            pl.BlockSpec((BT, D), lambda i: (i, 0)),
            pl.BlockSpec((D, D), lambda i: (0, 0)),
            pl.BlockSpec((D, D), lambda i: (0, 0)),
            pl.BlockSpec((1, D), lambda i: (0, 0)),
        ],
        out_specs=pl.BlockSpec((BT, D), lambda i: (i, 0)),
    )(agg, cnt, x, wlT, wrT, b)


def kernel(x, edge_index, W_l, b_l, W_r):
    N, D = x.shape
    E = edge_index.shape[1]

    ept = E // NTILES
    n_chunks = ept // CHUNK
    n_pad = -(-(N + 1) // (NSUB * 16)) * (NSUB * 16)

    agg, cnt = _sc_aggregate(x, edge_index.reshape(2 * E), n_pad, n_chunks)
    return _tc_combine(agg, (cnt[0] + cnt[1]).reshape(n_pad, 1), x,
                       W_l.T, W_r.T, b_l.reshape(1, D))


# speedup vs baseline: 1.0913x; 1.0182x over previous
import jax
import jax.numpy as jnp
from jax import lax
from jax.experimental import pallas as pl
from jax.experimental.pallas import tpu as pltpu
from jax.experimental.pallas import tpu_sc as plsc

NCORES = 2
NSUB = 16
NTILES = NCORES * NSUB
CHUNK = 80


def _sc_aggregate(x, ei, n_pad, n_chunks):
    D = x.shape[1]
    rows_per_tile = n_pad // NSUB
    n_zero_copies = rows_per_tile // CHUNK
    zero_rem = rows_per_tile % CHUNK

    def body(x_hbm, ei_hbm, agg_out, cnt_out,
             acc, cnta, idx_c, rows, ones_v, zc, isem, gsem, csem, ssem):
        c = lax.axis_index("c")
        s = lax.axis_index("s")
        w = c * NSUB + s
        ept = n_chunks * CHUNK
        n_edges = NTILES * ept
        ebase = w * ept

        def fetch_idx(j, slot):
            off = ebase + j * CHUNK
            pltpu.async_copy(ei_hbm.at[pl.ds(off, CHUNK)],
                             idx_c.at[slot, 0], isem)
            pltpu.async_copy(ei_hbm.at[pl.ds(n_edges + off, CHUNK)],
                             idx_c.at[slot, 1], isem)

        def wait_idx():
            pltpu.make_async_copy(ei_hbm.at[pl.ds(0, CHUNK)],
                                  idx_c.at[0, 0], isem).wait()
            pltpu.make_async_copy(ei_hbm.at[pl.ds(0, CHUNK)],
                                  idx_c.at[0, 1], isem).wait()

        def fill_rows(i, carry):
            r = i // (D // 16)
            k = i % (D // 16)
            rows[2, r, pl.ds(k * 16, 16)] = jnp.zeros((16,), jnp.float32)
            return carry

        lax.fori_loop(0, CHUNK * (D // 16), fill_rows, 0)

        def fill_ones(i, carry):
            ones_v[pl.ds(i * 16, 16)] = jnp.ones((16,), jnp.float32)
            return carry

        lax.fori_loop(0, CHUNK // 16, fill_ones, 0)

        def fill_zc(i, carry):
            zc[pl.ds(i * 16, 16)] = jnp.zeros((16,), jnp.float32)
            return carry

        lax.fori_loop(0, rows_per_tile // 16, fill_zc, 0)

        fetch_idx(0, 0)
        fetch_idx(1, 1)
        fetch_idx(2, 2)
        base = s * rows_per_tile
        for k in range(n_zero_copies):
            pltpu.async_copy(rows.at[2], acc.at[pl.ds(base + k * CHUNK, CHUNK)],
                             ssem)
        if zero_rem:
            pltpu.async_copy(rows.at[2, pl.ds(0, zero_rem)],
                             acc.at[pl.ds(base + n_zero_copies * CHUNK, zero_rem)],
                             ssem)
        pltpu.async_copy(zc, cnta.at[pl.ds(base, rows_per_tile)], ssem)
        wait_idx()
        pltpu.async_copy(x_hbm.at[idx_c.at[0, 0]], rows.at[0], gsem)
        wait_idx()
        pltpu.async_copy(x_hbm.at[idx_c.at[1, 0]], rows.at[1], gsem)
        for k in range(n_zero_copies):
            pltpu.make_async_copy(rows.at[2], acc.at[pl.ds(0, CHUNK)], ssem).wait()
        if zero_rem:
            pltpu.make_async_copy(rows.at[2, pl.ds(0, zero_rem)],
                                  acc.at[pl.ds(0, zero_rem)], ssem).wait()
        pltpu.make_async_copy(zc, cnta.at[pl.ds(0, rows_per_tile)], ssem).wait()
        plsc.subcore_barrier()

        def wait_scatter():
            pltpu.make_async_copy(rows.at[0], acc.at[idx_c.at[0, 1]], ssem).wait()

        def wait_count():
            pltpu.make_async_copy(ones_v, cnta.at[idx_c.at[0, 1]], csem).wait()

        def chunk_body(j, carry):
            buf = lax.rem(j, 3)
            sl = lax.rem(j, 4)
            gbuf = lax.rem(j + 2, 3)
            gsl = lax.rem(j + 2, 4)
            psl = lax.rem(j + 3, 4)

            @pl.when(j >= 1)
            def _():
                wait_scatter()
                wait_count()

            @pl.when(j + 3 < n_chunks)
            def _():
                fetch_idx(j + 3, psl)

            @pl.when(j + 2 < n_chunks)
            def _():
                wait_idx()
                pltpu.async_copy(x_hbm.at[idx_c.at[gsl, 0]], rows.at[gbuf], gsem)

            pltpu.make_async_copy(x_hbm.at[idx_c.at[sl, 0]], rows.at[buf], gsem).wait()
            pltpu.async_copy(ones_v, cnta.at[idx_c.at[sl, 1]], csem, add=True)
            pltpu.async_copy(rows.at[buf], acc.at[idx_c.at[sl, 1]], ssem, add=True)
            return carry

        lax.fori_loop(0, n_chunks - 1, chunk_body, 0)

        jl = n_chunks - 1
        buf = (jl) % 3
        sl = (jl) % 4
        wait_scatter()
        wait_count()
        pltpu.make_async_copy(x_hbm.at[idx_c.at[sl, 0]], rows.at[buf], gsem).wait()
        pltpu.async_copy(ones_v, cnta.at[idx_c.at[sl, 1]], csem, add=True)
        pltpu.async_copy(rows.at[buf], acc.at[idx_c.at[sl, 1]], ssem, add=True)
        wait_scatter()
        wait_count()
        plsc.subcore_barrier()

        wb = pltpu.async_copy(acc.at[pl.ds(base, rows_per_tile)],
                              agg_out.at[c, pl.ds(base, rows_per_tile)], gsem)
        wc = pltpu.async_copy(cnta.at[pl.ds(base, rows_per_tile)],
                              cnt_out.at[c, pl.ds(base, rows_per_tile)], csem)
        wb.wait()
        wc.wait()

    run = pl.kernel(
        body,
        out_type=(jax.ShapeDtypeStruct((NCORES, n_pad, D), jnp.float32),
                  jax.ShapeDtypeStruct((NCORES, n_pad), jnp.float32)),
        mesh=plsc.VectorSubcoreMesh(core_axis_name="c", subcore_axis_name="s"),
        scratch_types=(
            pltpu.VMEM_SHARED((n_pad, D), jnp.float32),
            pltpu.VMEM_SHARED((n_pad,), jnp.float32),
            pltpu.VMEM((4, 2, CHUNK), jnp.int32),
            pltpu.VMEM((3, CHUNK, D), jnp.float32),
            pltpu.VMEM((CHUNK,), jnp.float32),
            pltpu.VMEM((n_pad // NSUB,), jnp.float32),
            pltpu.SemaphoreType.DMA,
            pltpu.SemaphoreType.DMA,
            pltpu.SemaphoreType.DMA,
            pltpu.SemaphoreType.DMA,
        ),
    )
    return run(x, ei)


def _tc_combine(agg, cnt, x, wlT, wrT, b):
    N, D = x.shape
    BT = 1000

    def body(agg_ref, cnt_ref, x_ref, wl_ref, wr_ref, b_ref, out_ref):
        p = agg_ref[0] + agg_ref[1]
        cnt_col = cnt_ref[...]
        mean = p / jnp.maximum(cnt_col, 1.0)
        acc = jnp.dot(mean, wl_ref[...], preferred_element_type=jnp.float32)
        acc = acc + jnp.dot(x_ref[...], wr_ref[...], preferred_element_type=jnp.float32)
        out_ref[...] = jnp.maximum(acc + b_ref[...], 0.0)

    return pl.pallas_call(
        body,
        out_shape=jax.ShapeDtypeStruct((N, D), jnp.float32),
        grid=(N // BT,),
        in_specs=[
            pl.BlockSpec((NCORES, BT, D), lambda i: (0, i, 0)),
            pl.BlockSpec((BT, 1), lambda i: (i, 0)),
            pl.BlockSpec((BT, D), lambda i: (i, 0)),
            pl.BlockSpec((D, D), lambda i: (0, 0)),
            pl.BlockSpec((D, D), lambda i: (0, 0)),
            pl.BlockSpec((1, D), lambda i: (0, 0)),
        ],
        out_specs=pl.BlockSpec((BT, D), lambda i: (i, 0)),
    )(agg, cnt, x, wlT, wrT, b)


def kernel(x, edge_index, W_l, b_l, W_r):
    N, D = x.shape
    E = edge_index.shape[1]

    ept = E // NTILES
    n_chunks = ept // CHUNK
    n_pad = -(-(N + 1) // (NSUB * 16)) * (NSUB * 16)

    agg, cnt = _sc_aggregate(x, edge_index.reshape(2 * E), n_pad, n_chunks)
    return _tc_combine(agg, (cnt[0] + cnt[1]).reshape(n_pad, 1), x,
                       W_l.T, W_r.T, b_l.reshape(1, D))
